# Initial kernel scaffold; baseline (speedup 1.0000x reference)
import jax
import jax.numpy as jnp
from jax.experimental import pallas as pl


def kernel(x, edge_index, W0, b0, W1, b1, W2, b2, W3, b3, W_last, b_last):
    raise NotImplementedError("write your pallas kernel here")


# speedup vs baseline: 14.0954x; 14.0954x over previous
import functools

import jax
import jax.numpy as jnp
from jax import lax
from jax.experimental import pallas as pl
from jax.experimental.pallas import tpu as pltpu
from jax.experimental.pallas import tpu_sc as plsc

N = 10000
D = 128
NCLS = 64
NE = 320000
NL = 4

NC = 2
NS = 16
NW = NC * NS
EPT = NE // NW
K = 80
CH = EPT // K

NPAD = 10240
NPAD2 = 16384
HR = NPAD2 // 128
HRT = HR // NS
RP = NPAD // NS
CPR = 80
NCP = RP // CPR

RB = 1024


def _sc_mesh():
  return plsc.VectorSubcoreMesh(core_axis_name="c", subcore_axis_name="s")


def _degree(dst1, iota2):
  @functools.partial(
      pl.kernel,
      out_type=jax.ShapeDtypeStruct((NC, HR, 128), jnp.float32),
      mesh=_sc_mesh(),
      scratch_types=[
          pltpu.VMEM((EPT,), jnp.int32),
          pltpu.VMEM((NPAD2,), jnp.float32),
          pltpu.VMEM((HR, 128), jnp.float32),
          pltpu.VMEM((1, 128), jnp.int32),
          pltpu.VMEM((HRT, 128), jnp.float32),
          pltpu.VMEM_SHARED((HR, 128), jnp.float32),
      ],
      compiler_params=pltpu.CompilerParams(needs_layout_passes=False),
  )
  def deg_kernel(dst_hbm, iota_hbm, out_hbm, dst_v, hist1, hist, iota_v,
                 bounce, acc):
    c = lax.axis_index("c")
    s = lax.axis_index("s")
    w = s * NC + c

    pltpu.sync_copy(dst_hbm.at[w], dst_v)
    pltpu.sync_copy(iota_hbm, iota_v)

    def zero_hist(i, _):
      hist1[pl.ds(i * 16, 16)] = jnp.zeros((16,), jnp.float32)
      return 0
    lax.fori_loop(0, NPAD2 // 16, zero_hist, 0)

    def zero_bounce(i, _):
      r = i // 8
      t = i % 8
      bounce[r, pl.ds(t * 16, 16)] = jnp.zeros((16,), jnp.float32)
      return 0
    lax.fori_loop(0, HRT * 8, zero_bounce, 0)

    pltpu.sync_copy(bounce, acc.at[pl.ds(s * HRT, HRT)])

    ones = jnp.ones((16,), jnp.float32)

    def hist_body(j, _):
      idx = dst_v[pl.ds(j * 16, 16)]
      plsc.addupdate_scatter(hist1, [idx], ones)
      return 0
    lax.fori_loop(0, EPT // 16, hist_body, 0)

    def repack(i, _):
      r = i // 8
      t = i % 8
      hist[r, pl.ds(t * 16, 16)] = hist1[pl.ds(r * 128 + t * 16, 16)]
      return 0
    lax.fori_loop(0, HR * 8, repack, 0)

    plsc.subcore_barrier()

    pltpu.sync_copy(hist, acc.at[iota_v.at[0]], add=True)

    plsc.subcore_barrier()

    pltpu.sync_copy(acc.at[pl.ds(s * HRT, HRT)], bounce)
    pltpu.sync_copy(bounce, out_hbm.at[c, pl.ds(s * HRT, HRT)])

  return deg_kernel(dst1, iota2)


def _propagate(g, src1, dst3):
  @functools.partial(
      pl.kernel,
      out_type=jax.ShapeDtypeStruct((NC, NPAD, D), jnp.float32),
      mesh=_sc_mesh(),
      scratch_types=[
          pltpu.VMEM((EPT,), jnp.int32),
          pltpu.VMEM((CH, K), jnp.int32),
          pltpu.VMEM((K, D), jnp.float32),
          pltpu.VMEM((K, D), jnp.float32),
          pltpu.VMEM_SHARED((NPAD, D), jnp.float32),
          pltpu.SemaphoreType.DMA,
      ],
  )
  def prop_kernel(g_hbm, src_hbm, dst_hbm, out_hbm, src_v, dst_v, rows_a,
                  rows_b, acc, sem):
    c = lax.axis_index("c")
    s = lax.axis_index("s")
    w = s * NC + c
    base = s * RP

    pltpu.sync_copy(src_hbm.at[w], src_v)
    pltpu.sync_copy(dst_hbm.at[w], dst_v)

    @pl.when(c == 0)
    def _():
      for i in range(NCP):
        pltpu.sync_copy(g_hbm.at[pl.ds(base + i * CPR, CPR)], rows_b)
        pltpu.sync_copy(rows_b, acc.at[pl.ds(base + i * CPR, CPR)])

    @pl.when(c != 0)
    def _():
      def zero_row(i, _):
        r = i // (D // 16)
        t = i % (D // 16)
        rows_b[r, pl.ds(t * 16, 16)] = jnp.zeros((16,), jnp.float32)
        return 0
      lax.fori_loop(0, CPR * (D // 16), zero_row, 0)
      for i in range(NCP):
        pltpu.sync_copy(rows_b, acc.at[pl.ds(base + i * CPR, CPR)])

    plsc.subcore_barrier()

    def chunk(j, _):
      pltpu.async_copy(g_hbm.at[src_v.at[pl.ds(j * K, K)]], rows_a, sem).wait()
      pltpu.sync_copy(rows_a, acc.at[dst_v.at[j]], add=True)
      return 0
    lax.fori_loop(0, CH, chunk, 0)

    plsc.subcore_barrier()

    for i in range(NCP):
      pltpu.sync_copy(acc.at[pl.ds(base + i * CPR, CPR)], rows_b)
      pltpu.sync_copy(rows_b, out_hbm.at[c, pl.ds(base + i * CPR, CPR)])

  return prop_kernel(g, src1, dst3)


def _row_spec(width):
  return pl.BlockSpec((RB, width), lambda i: (i, 0))


def _full_spec(r, cdim):
  return pl.BlockSpec((r, cdim), lambda i: (0, 0))


def _first_body(x_ref, w_ref, d0_ref, d1_ref, g_ref):
  dinv = lax.rsqrt(d0_ref[...] + d1_ref[...] + 1.0)
  g_ref[...] = dinv * jnp.dot(x_ref[...], w_ref[...],
                              preferred_element_type=jnp.float32)


def _tc_first(x, W0, d0, d1):
  return pl.pallas_call(
      _first_body,
      grid=(NPAD // RB,),
      in_specs=[_row_spec(D), _full_spec(D, D), _row_spec(1), _row_spec(1)],
      out_specs=_row_spec(D),
      out_shape=jax.ShapeDtypeStruct((NPAD, D), jnp.float32),
  )(x, W0, d0, d1)


def _mid_body(s0_ref, s1_ref, d0_ref, d1_ref, b_ref, wn_ref, wh_ref, y_ref,
              g_ref, yo_ref):
  dinv = lax.rsqrt(d0_ref[...] + d1_ref[...] + 1.0)
  h = jnp.maximum(dinv * (s0_ref[...] + s1_ref[...]) + b_ref[...], 0.0)
  g_ref[...] = dinv * jnp.dot(h, wn_ref[...],
                              preferred_element_type=jnp.float32)
  yo_ref[...] = y_ref[...] + jnp.dot(h, wh_ref[...],
                                     preferred_element_type=jnp.float32)


def _tc_mid(s0, s1, d0, d1, b, Wn, Wh, y):
  return pl.pallas_call(
      _mid_body,
      grid=(NPAD // RB,),
      in_specs=[_row_spec(D), _row_spec(D), _row_spec(1), _row_spec(1),
                _full_spec(1, D), _full_spec(D, D), _full_spec(D, NCLS),
                _row_spec(NCLS)],
      out_specs=[_row_spec(D), _row_spec(NCLS)],
      out_shape=[jax.ShapeDtypeStruct((NPAD, D), jnp.float32),
                 jax.ShapeDtypeStruct((NPAD, NCLS), jnp.float32)],
  )(s0, s1, d0, d1, b, Wn, Wh, y)


def _last_body(s0_ref, s1_ref, d0_ref, d1_ref, b_ref, wh_ref, y_ref, yo_ref):
  dinv = lax.rsqrt(d0_ref[...] + d1_ref[...] + 1.0)
  h = jnp.maximum(dinv * (s0_ref[...] + s1_ref[...]) + b_ref[...], 0.0)
  yo_ref[...] = y_ref[...] + jnp.dot(h, wh_ref[...],
                                     preferred_element_type=jnp.float32)


def _tc_last(s0, s1, d0, d1, b, Wh, y):
  return pl.pallas_call(
      _last_body,
      grid=(NPAD // RB,),
      in_specs=[_row_spec(D), _row_spec(D), _row_spec(1), _row_spec(1),
                _full_spec(1, D), _full_spec(D, NCLS), _row_spec(NCLS)],
      out_specs=_row_spec(NCLS),
      out_shape=jax.ShapeDtypeStruct((NPAD, NCLS), jnp.float32),
  )(s0, s1, d0, d1, b, Wh, y)


def kernel(x, edge_index, W0, b0, W1, b1, W2, b2, W3, b3, W_last, b_last):
  src1 = edge_index[0].reshape(NW, EPT)
  dst3 = edge_index[1].reshape(NW, CH, K)
  dst1 = edge_index[1].reshape(NW, EPT)

  iota2 = jnp.arange(HR, dtype=jnp.int32).reshape(1, HR)

  degp = _degree(dst1, iota2)
  degf = degp.reshape(NC, NPAD2)[:, :NPAD]
  d0 = degf[0].reshape(NPAD, 1)
  d1 = degf[1].reshape(NPAD, 1)

  Ws = [W0, W1, W2, W3]
  bs = [b.reshape(1, D) for b in (b0, b1, b2, b3)]
  Whs = [W_last[l * D:(l + 1) * D] for l in range(NL)]

  x_pad = jnp.zeros((NPAD, D), jnp.float32).at[:N].set(x)
  y = jnp.broadcast_to(b_last.reshape(1, NCLS), (NPAD, NCLS))

  g = _tc_first(x_pad, Ws[0], d0, d1)
  for l in range(NL):
    sp = _propagate(g, src1, dst3)
    if l < NL - 1:
      g, y = _tc_mid(sp[0], sp[1], d0, d1, bs[l], Ws[l + 1], Whs[l], y)
    else:
      y = _tc_last(sp[0], sp[1], d0, d1, bs[l], Whs[l], y)
  return y[:N]


# speedup vs baseline: 22.3388x; 1.5848x over previous
import functools

import jax
import jax.numpy as jnp
from jax import lax
from jax.experimental import pallas as pl
from jax.experimental.pallas import tpu as pltpu
from jax.experimental.pallas import tpu_sc as plsc

N = 10000
D = 128
NCLS = 64
NE = 320000
NL = 4

NC = 2
NS = 16
NW = NC * NS
EPT = NE // NW
K = 80
CH = EPT // K

NPAD = 10240
NPAD2 = 16384
HR = NPAD2 // 128
HRT = HR // NS
RP = NPAD // NS
CPR = 80
NCP = RP // CPR

RB = 1024


def _sc_mesh():
  return plsc.VectorSubcoreMesh(core_axis_name="c", subcore_axis_name="s")


def _degree(dst1, iota2):
  @functools.partial(
      pl.kernel,
      out_type=jax.ShapeDtypeStruct((NC, HR, 128), jnp.float32),
      mesh=_sc_mesh(),
      scratch_types=[
          pltpu.VMEM((EPT,), jnp.int32),
          pltpu.VMEM((NPAD2,), jnp.float32),
          pltpu.VMEM((HR, 128), jnp.float32),
          pltpu.VMEM((1, 128), jnp.int32),
          pltpu.VMEM((HRT, 128), jnp.float32),
          pltpu.VMEM_SHARED((HR, 128), jnp.float32),
      ],
      compiler_params=pltpu.CompilerParams(needs_layout_passes=False),
  )
  def deg_kernel(dst_hbm, iota_hbm, out_hbm, dst_v, hist1, hist, iota_v,
                 bounce, acc):
    c = lax.axis_index("c")
    s = lax.axis_index("s")
    w = s * NC + c

    pltpu.sync_copy(dst_hbm.at[w], dst_v)
    pltpu.sync_copy(iota_hbm, iota_v)

    def zero_hist(i, _):
      hist1[pl.ds(i * 16, 16)] = jnp.zeros((16,), jnp.float32)
      return 0
    lax.fori_loop(0, NPAD2 // 16, zero_hist, 0)

    def zero_bounce(i, _):
      r = i // 8
      t = i % 8
      bounce[r, pl.ds(t * 16, 16)] = jnp.zeros((16,), jnp.float32)
      return 0
    lax.fori_loop(0, HRT * 8, zero_bounce, 0)

    pltpu.sync_copy(bounce, acc.at[pl.ds(s * HRT, HRT)])

    ones = jnp.ones((16,), jnp.float32)

    def hist_body(j, _):
      idx = dst_v[pl.ds(j * 16, 16)]
      plsc.addupdate_scatter(hist1, [idx], ones)
      return 0
    lax.fori_loop(0, EPT // 16, hist_body, 0)

    def repack(i, _):
      r = i // 8
      t = i % 8
      hist[r, pl.ds(t * 16, 16)] = hist1[pl.ds(r * 128 + t * 16, 16)]
      return 0
    lax.fori_loop(0, HR * 8, repack, 0)

    plsc.subcore_barrier()

    pltpu.sync_copy(hist, acc.at[iota_v.at[0]], add=True)

    plsc.subcore_barrier()

    pltpu.sync_copy(acc.at[pl.ds(s * HRT, HRT)], bounce)
    pltpu.sync_copy(bounce, out_hbm.at[c, pl.ds(s * HRT, HRT)])

  return deg_kernel(dst1, iota2)


def _propagate(g, src1, dst3):
  @functools.partial(
      pl.kernel,
      out_type=jax.ShapeDtypeStruct((NC, NPAD, D), jnp.float32),
      mesh=_sc_mesh(),
      scratch_types=[
          pltpu.VMEM((EPT,), jnp.int32),
          pltpu.VMEM((CH, K), jnp.int32),
          pltpu.VMEM((K, D), jnp.float32),
          pltpu.VMEM((K, D), jnp.float32),
          pltpu.VMEM_SHARED((NPAD, D), jnp.float32),
          pltpu.SemaphoreType.DMA,
          pltpu.SemaphoreType.DMA,
      ],
  )
  def prop_kernel(g_hbm, src_hbm, dst_hbm, out_hbm, src_v, dst_v, rows_a,
                  rows_b, acc, sem_a, sem_b):
    c = lax.axis_index("c")
    s = lax.axis_index("s")
    w = s * NC + c
    base = s * RP

    pltpu.sync_copy(src_hbm.at[w], src_v)
    pltpu.sync_copy(dst_hbm.at[w], dst_v)

    pltpu.async_copy(g_hbm.at[src_v.at[pl.ds(0, K)]], rows_a, sem_a)

    def zero_row(i, _):
      r = i // (D // 16)
      t = i % (D // 16)
      rows_b[r, pl.ds(t * 16, 16)] = jnp.zeros((16,), jnp.float32)
      return 0
    lax.fori_loop(0, CPR * (D // 16), zero_row, 0)
    for i in range(NCP):
      pltpu.sync_copy(rows_b, acc.at[pl.ds(base + i * CPR, CPR)])

    plsc.subcore_barrier()

    def chunk2(i, _):
      ja = 2 * i
      jb = 2 * i + 1
      pltpu.async_copy(g_hbm.at[src_v.at[pl.ds(jb * K, K)]], rows_b, sem_b)
      pltpu.make_async_copy(
          g_hbm.at[src_v.at[pl.ds(ja * K, K)]], rows_a, sem_a).wait()
      pltpu.sync_copy(rows_a, acc.at[dst_v.at[ja]], add=True)
      pltpu.async_copy(
          g_hbm.at[src_v.at[pl.ds((ja + 2) * K, K)]], rows_a, sem_a)
      pltpu.make_async_copy(
          g_hbm.at[src_v.at[pl.ds(jb * K, K)]], rows_b, sem_b).wait()
      pltpu.sync_copy(rows_b, acc.at[dst_v.at[jb]], add=True)
      return 0
    lax.fori_loop(0, (CH - 1) // 2, chunk2, 0)

    pltpu.make_async_copy(
        g_hbm.at[src_v.at[pl.ds((CH - 1) * K, K)]], rows_a, sem_a).wait()
    pltpu.sync_copy(rows_a, acc.at[dst_v.at[CH - 1]], add=True)

    plsc.subcore_barrier()

    for i in range(NCP):
      pltpu.sync_copy(acc.at[pl.ds(base + i * CPR, CPR)], rows_b)
      pltpu.sync_copy(rows_b, out_hbm.at[c, pl.ds(base + i * CPR, CPR)])

  return prop_kernel(g, src1, dst3)


def _row_spec(width):
  return pl.BlockSpec((RB, width), lambda i: (i, 0))


def _full_spec(r, cdim):
  return pl.BlockSpec((r, cdim), lambda i: (0, 0))


def _first_body(x_ref, w_ref, d0_ref, d1_ref, g_ref):
  dinv = lax.rsqrt(d0_ref[...] + d1_ref[...] + 1.0)
  g_ref[...] = dinv * jnp.dot(x_ref[...], w_ref[...],
                              preferred_element_type=jnp.float32)


def _tc_first(x, W0, d0, d1):
  return pl.pallas_call(
      _first_body,
      grid=(NPAD // RB,),
      in_specs=[_row_spec(D), _full_spec(D, D), _row_spec(1), _row_spec(1)],
      out_specs=_row_spec(D),
      out_shape=jax.ShapeDtypeStruct((NPAD, D), jnp.float32),
  )(x, W0, d0, d1)


def _mid_body(s0_ref, s1_ref, g_in_ref, d0_ref, d1_ref, b_ref, wn_ref, wh_ref,
              y_ref, g_ref, yo_ref):
  dinv = lax.rsqrt(d0_ref[...] + d1_ref[...] + 1.0)
  h = jnp.maximum(
      dinv * (s0_ref[...] + s1_ref[...] + g_in_ref[...]) + b_ref[...], 0.0)
  g_ref[...] = dinv * jnp.dot(h, wn_ref[...],
                              preferred_element_type=jnp.float32)
  yo_ref[...] = y_ref[...] + jnp.dot(h, wh_ref[...],
                                     preferred_element_type=jnp.float32)


def _tc_mid(s0, s1, g_in, d0, d1, b, Wn, Wh, y):
  return pl.pallas_call(
      _mid_body,
      grid=(NPAD // RB,),
      in_specs=[_row_spec(D), _row_spec(D), _row_spec(D), _row_spec(1),
                _row_spec(1), _full_spec(1, D), _full_spec(D, D),
                _full_spec(D, NCLS), _row_spec(NCLS)],
      out_specs=[_row_spec(D), _row_spec(NCLS)],
      out_shape=[jax.ShapeDtypeStruct((NPAD, D), jnp.float32),
                 jax.ShapeDtypeStruct((NPAD, NCLS), jnp.float32)],
  )(s0, s1, g_in, d0, d1, b, Wn, Wh, y)


def _last_body(s0_ref, s1_ref, g_in_ref, d0_ref, d1_ref, b_ref, wh_ref, y_ref,
               yo_ref):
  dinv = lax.rsqrt(d0_ref[...] + d1_ref[...] + 1.0)
  h = jnp.maximum(
      dinv * (s0_ref[...] + s1_ref[...] + g_in_ref[...]) + b_ref[...], 0.0)
  yo_ref[...] = y_ref[...] + jnp.dot(h, wh_ref[...],
                                     preferred_element_type=jnp.float32)


def _tc_last(s0, s1, g_in, d0, d1, b, Wh, y):
  return pl.pallas_call(
      _last_body,
      grid=(NPAD // RB,),
      in_specs=[_row_spec(D), _row_spec(D), _row_spec(D), _row_spec(1),
                _row_spec(1), _full_spec(1, D), _full_spec(D, NCLS),
                _row_spec(NCLS)],
      out_specs=_row_spec(NCLS),
      out_shape=jax.ShapeDtypeStruct((NPAD, NCLS), jnp.float32),
  )(s0, s1, g_in, d0, d1, b, Wh, y)


def kernel(x, edge_index, W0, b0, W1, b1, W2, b2, W3, b3, W_last, b_last):
  src1 = edge_index[0].reshape(NW, EPT)
  dst3 = edge_index[1].reshape(NW, CH, K)
  dst1 = edge_index[1].reshape(NW, EPT)

  iota2 = jnp.arange(HR, dtype=jnp.int32).reshape(1, HR)

  degp = _degree(dst1, iota2)
  degf = degp.reshape(NC, NPAD2)[:, :NPAD]
  d0 = degf[0].reshape(NPAD, 1)
  d1 = degf[1].reshape(NPAD, 1)

  Ws = [W0, W1, W2, W3]
  bs = [b.reshape(1, D) for b in (b0, b1, b2, b3)]
  Whs = [W_last[l * D:(l + 1) * D] for l in range(NL)]

  x_pad = jnp.zeros((NPAD, D), jnp.float32).at[:N].set(x)
  y = jnp.broadcast_to(b_last.reshape(1, NCLS), (NPAD, NCLS))

  g = _tc_first(x_pad, Ws[0], d0, d1)
  for l in range(NL):
    sp = _propagate(g, src1, dst3)
    if l < NL - 1:
      g, y = _tc_mid(sp[0], sp[1], g, d0, d1, bs[l], Ws[l + 1], Whs[l], y)
    else:
      y = _tc_last(sp[0], sp[1], g, d0, d1, bs[l], Whs[l], y)
  return y[:N]


# speedup vs baseline: 23.7762x; 1.0643x over previous
import functools

import jax
import jax.numpy as jnp
from jax import lax
from jax.experimental import pallas as pl
from jax.experimental.pallas import tpu as pltpu
from jax.experimental.pallas import tpu_sc as plsc

N = 10000
D = 128
NCLS = 64
NE = 320000
NL = 4

NC = 2
NS = 16
NW = NC * NS
EPT = NE // NW
K = 80
CH = EPT // K

NPAD = 10240
NPAD2 = 16384
HR = NPAD2 // 128
HRT = HR // NS
RP = NPAD // NS
CPR = 80
NCP = RP // CPR

RB = 1024


def _sc_mesh():
  return plsc.VectorSubcoreMesh(core_axis_name="c", subcore_axis_name="s")


def _degree(dst1, iota2):
  @functools.partial(
      pl.kernel,
      out_type=jax.ShapeDtypeStruct((NC, HR, 128), jnp.float32),
      mesh=_sc_mesh(),
      scratch_types=[
          pltpu.VMEM((EPT,), jnp.int32),
          pltpu.VMEM((NPAD2,), jnp.float32),
          pltpu.VMEM((HR, 128), jnp.float32),
          pltpu.VMEM((1, 128), jnp.int32),
          pltpu.VMEM((HRT, 128), jnp.float32),
          pltpu.VMEM_SHARED((HR, 128), jnp.float32),
      ],
      compiler_params=pltpu.CompilerParams(needs_layout_passes=False),
  )
  def deg_kernel(dst_hbm, iota_hbm, out_hbm, dst_v, hist1, hist, iota_v,
                 bounce, acc):
    c = lax.axis_index("c")
    s = lax.axis_index("s")
    w = s * NC + c

    pltpu.sync_copy(dst_hbm.at[w], dst_v)
    pltpu.sync_copy(iota_hbm, iota_v)

    def zero_hist(i, _):
      hist1[pl.ds(i * 16, 16)] = jnp.zeros((16,), jnp.float32)
      return 0
    lax.fori_loop(0, NPAD2 // 16, zero_hist, 0)

    def zero_bounce(i, _):
      r = i // 8
      t = i % 8
      bounce[r, pl.ds(t * 16, 16)] = jnp.zeros((16,), jnp.float32)
      return 0
    lax.fori_loop(0, HRT * 8, zero_bounce, 0)

    pltpu.sync_copy(bounce, acc.at[pl.ds(s * HRT, HRT)])

    ones = jnp.ones((16,), jnp.float32)

    def hist_body(j, _):
      idx = dst_v[pl.ds(j * 16, 16)]
      plsc.addupdate_scatter(hist1, [idx], ones)
      return 0
    lax.fori_loop(0, EPT // 16, hist_body, 0)

    def repack(i, _):
      r = i // 8
      t = i % 8
      hist[r, pl.ds(t * 16, 16)] = hist1[pl.ds(r * 128 + t * 16, 16)]
      return 0
    lax.fori_loop(0, HR * 8, repack, 0)

    plsc.subcore_barrier()

    pltpu.sync_copy(hist, acc.at[iota_v.at[0]], add=True)

    plsc.subcore_barrier()

    pltpu.sync_copy(acc.at[pl.ds(s * HRT, HRT)], bounce)
    pltpu.sync_copy(bounce, out_hbm.at[c, pl.ds(s * HRT, HRT)])

  return deg_kernel(dst1, iota2)


def _propagate(g, src1, dst3):
  @functools.partial(
      pl.kernel,
      out_type=(jax.ShapeDtypeStruct((NPAD, D), jnp.float32),
                jax.ShapeDtypeStruct((NPAD, D), jnp.float32)),
      mesh=_sc_mesh(),
      scratch_types=[
          pltpu.VMEM((EPT,), jnp.int32),
          pltpu.VMEM((CH, K), jnp.int32),
          pltpu.VMEM((K, D), jnp.float32),
          pltpu.VMEM((K, D), jnp.float32),
          pltpu.VMEM_SHARED((NPAD, D), jnp.float32),
          pltpu.SemaphoreType.DMA,
          pltpu.SemaphoreType.DMA,
      ],
  )
  def prop_kernel(g_hbm, src_hbm, dst_hbm, out0_hbm, out1_hbm, src_v, dst_v,
                  rows_a, rows_b, acc, sem_a, sem_b):
    c = lax.axis_index("c")
    s = lax.axis_index("s")
    w = s * NC + c
    base = s * RP

    pltpu.sync_copy(src_hbm.at[w], src_v)
    pltpu.sync_copy(dst_hbm.at[w], dst_v)

    pltpu.async_copy(g_hbm.at[src_v.at[pl.ds(0, K)]], rows_a, sem_a)

    def zero_row(i, _):
      r = i // (D // 16)
      t = i % (D // 16)
      rows_b[r, pl.ds(t * 16, 16)] = jnp.zeros((16,), jnp.float32)
      return 0
    lax.fori_loop(0, CPR * (D // 16), zero_row, 0)
    for i in range(NCP):
      pltpu.async_copy(rows_b, acc.at[pl.ds(base + i * CPR, CPR)], sem_b)
    for i in range(NCP):
      pltpu.make_async_copy(
          rows_b, acc.at[pl.ds(base + i * CPR, CPR)], sem_b).wait()

    plsc.subcore_barrier()

    def chunk2(i, _):
      ja = 2 * i
      jb = 2 * i + 1
      pltpu.async_copy(g_hbm.at[src_v.at[pl.ds(jb * K, K)]], rows_b, sem_b)
      pltpu.make_async_copy(
          g_hbm.at[src_v.at[pl.ds(ja * K, K)]], rows_a, sem_a).wait()
      pltpu.sync_copy(rows_a, acc.at[dst_v.at[ja]], add=True)
      pltpu.async_copy(
          g_hbm.at[src_v.at[pl.ds((ja + 2) * K, K)]], rows_a, sem_a)
      pltpu.make_async_copy(
          g_hbm.at[src_v.at[pl.ds(jb * K, K)]], rows_b, sem_b).wait()
      pltpu.sync_copy(rows_b, acc.at[dst_v.at[jb]], add=True)
      return 0
    lax.fori_loop(0, (CH - 1) // 2, chunk2, 0)

    pltpu.make_async_copy(
        g_hbm.at[src_v.at[pl.ds((CH - 1) * K, K)]], rows_a, sem_a).wait()
    pltpu.sync_copy(rows_a, acc.at[dst_v.at[CH - 1]], add=True)

    plsc.subcore_barrier()

    def copy_out(out_hbm):
      for i in range(NCP):
        buf = rows_a if i % 2 == 0 else rows_b
        sem = sem_a if i % 2 == 0 else sem_b
        sl = pl.ds(base + i * CPR, CPR)
        if i >= 2:
          prev = pl.ds(base + (i - 2) * CPR, CPR)
          pltpu.make_async_copy(buf, out_hbm.at[prev], sem).wait()
        pltpu.sync_copy(acc.at[sl], buf)
        pltpu.async_copy(buf, out_hbm.at[sl], sem)
      for i in range(NCP - 2, NCP):
        buf = rows_a if i % 2 == 0 else rows_b
        sem = sem_a if i % 2 == 0 else sem_b
        sl = pl.ds(base + i * CPR, CPR)
        pltpu.make_async_copy(buf, out_hbm.at[sl], sem).wait()

    @pl.when(c == 0)
    def _():
      copy_out(out0_hbm)

    @pl.when(c != 0)
    def _():
      copy_out(out1_hbm)

  return prop_kernel(g, src1, dst3)


def _row_spec(width):
  return pl.BlockSpec((RB, width), lambda i: (i, 0))


def _full_spec(r, cdim):
  return pl.BlockSpec((r, cdim), lambda i: (0, 0))


def _first_body(x_ref, w_ref, d0_ref, d1_ref, g_ref):
  dinv = lax.rsqrt(d0_ref[...] + d1_ref[...] + 1.0)
  g_ref[...] = dinv * jnp.dot(x_ref[...], w_ref[...],
                              preferred_element_type=jnp.float32)


def _tc_first(x, W0, d0, d1):
  return pl.pallas_call(
      _first_body,
      grid=(NPAD // RB,),
      in_specs=[_row_spec(D), _full_spec(D, D), _row_spec(1), _row_spec(1)],
      out_specs=_row_spec(D),
      out_shape=jax.ShapeDtypeStruct((NPAD, D), jnp.float32),
  )(x, W0, d0, d1)


def _mid_body(s0_ref, s1_ref, g_in_ref, d0_ref, d1_ref, b_ref, wn_ref, wh_ref,
              y_ref, g_ref, yo_ref):
  dinv = lax.rsqrt(d0_ref[...] + d1_ref[...] + 1.0)
  h = jnp.maximum(
      dinv * (s0_ref[...] + s1_ref[...] + g_in_ref[...]) + b_ref[...], 0.0)
  g_ref[...] = dinv * jnp.dot(h, wn_ref[...],
                              preferred_element_type=jnp.float32)
  yo_ref[...] = y_ref[...] + jnp.dot(h, wh_ref[...],
                                     preferred_element_type=jnp.float32)


def _tc_mid(s0, s1, g_in, d0, d1, b, Wn, Wh, y):
  return pl.pallas_call(
      _mid_body,
      grid=(NPAD // RB,),
      in_specs=[_row_spec(D), _row_spec(D), _row_spec(D), _row_spec(1),
                _row_spec(1), _full_spec(1, D), _full_spec(D, D),
                _full_spec(D, NCLS), _row_spec(NCLS)],
      out_specs=[_row_spec(D), _row_spec(NCLS)],
      out_shape=[jax.ShapeDtypeStruct((NPAD, D), jnp.float32),
                 jax.ShapeDtypeStruct((NPAD, NCLS), jnp.float32)],
  )(s0, s1, g_in, d0, d1, b, Wn, Wh, y)


def _last_body(s0_ref, s1_ref, g_in_ref, d0_ref, d1_ref, b_ref, wh_ref, y_ref,
               yo_ref):
  dinv = lax.rsqrt(d0_ref[...] + d1_ref[...] + 1.0)
  h = jnp.maximum(
      dinv * (s0_ref[...] + s1_ref[...] + g_in_ref[...]) + b_ref[...], 0.0)
  yo_ref[...] = y_ref[...] + jnp.dot(h, wh_ref[...],
                                     preferred_element_type=jnp.float32)


def _tc_last(s0, s1, g_in, d0, d1, b, Wh, y):
  return pl.pallas_call(
      _last_body,
      grid=(NPAD // RB,),
      in_specs=[_row_spec(D), _row_spec(D), _row_spec(D), _row_spec(1),
                _row_spec(1), _full_spec(1, D), _full_spec(D, NCLS),
                _row_spec(NCLS)],
      out_specs=_row_spec(NCLS),
      out_shape=jax.ShapeDtypeStruct((NPAD, NCLS), jnp.float32),
  )(s0, s1, g_in, d0, d1, b, Wh, y)


def kernel(x, edge_index, W0, b0, W1, b1, W2, b2, W3, b3, W_last, b_last):
  src1 = edge_index[0].reshape(NW, EPT)
  dst3 = edge_index[1].reshape(NW, CH, K)
  dst1 = edge_index[1].reshape(NW, EPT)

  iota2 = jnp.arange(HR, dtype=jnp.int32).reshape(1, HR)

  degp = _degree(dst1, iota2)
  degf = degp.reshape(NC, NPAD2)[:, :NPAD]
  d0 = degf[0].reshape(NPAD, 1)
  d1 = degf[1].reshape(NPAD, 1)

  Ws = [W0, W1, W2, W3]
  bs = [b.reshape(1, D) for b in (b0, b1, b2, b3)]
  Whs = [W_last[l * D:(l + 1) * D] for l in range(NL)]

  x_pad = jnp.zeros((NPAD, D), jnp.float32).at[:N].set(x)
  y = jnp.broadcast_to(b_last.reshape(1, NCLS), (NPAD, NCLS))

  g = _tc_first(x_pad, Ws[0], d0, d1)
  for l in range(NL):
    s0, s1 = _propagate(g, src1, dst3)
    if l < NL - 1:
      g, y = _tc_mid(s0, s1, g, d0, d1, bs[l], Ws[l + 1], Whs[l], y)
    else:
      y = _tc_last(s0, s1, g, d0, d1, bs[l], Whs[l], y)
  return y[:N]


# speedup vs baseline: 24.0282x; 1.0106x over previous
import functools

import jax
import jax.numpy as jnp
from jax import lax
from jax.experimental import pallas as pl
from jax.experimental.pallas import tpu as pltpu
from jax.experimental.pallas import tpu_sc as plsc

N = 10000
D = 128
NCLS = 64
NE = 320000
NL = 4

NC = 2
NS = 16
NW = NC * NS
EPT = NE // NW
K = 80
CH = EPT // K

NPAD = 10240
NPAD2 = 16384
HR = NPAD2 // 128
HRT = HR // NS
RP = NPAD // NS
CPR = 80
NCP = RP // CPR

RB = 1024
RBL = 1000


def _sc_mesh():
  return plsc.VectorSubcoreMesh(core_axis_name="c", subcore_axis_name="s")


def _degree(dst1, iota2):
  @functools.partial(
      pl.kernel,
      out_type=jax.ShapeDtypeStruct((NC, HR, 128), jnp.float32),
      mesh=_sc_mesh(),
      scratch_types=[
          pltpu.VMEM((EPT,), jnp.int32),
          pltpu.VMEM((NPAD2,), jnp.float32),
          pltpu.VMEM((HR, 128), jnp.float32),
          pltpu.VMEM((1, 128), jnp.int32),
          pltpu.VMEM((HRT, 128), jnp.float32),
          pltpu.VMEM_SHARED((HR, 128), jnp.float32),
      ],
      compiler_params=pltpu.CompilerParams(needs_layout_passes=False),
  )
  def deg_kernel(dst_hbm, iota_hbm, out_hbm, dst_v, hist1, hist, iota_v,
                 bounce, acc):
    c = lax.axis_index("c")
    s = lax.axis_index("s")
    w = s * NC + c

    pltpu.sync_copy(dst_hbm.at[w], dst_v)
    pltpu.sync_copy(iota_hbm, iota_v)

    def zero_hist(i, _):
      hist1[pl.ds(i * 16, 16)] = jnp.zeros((16,), jnp.float32)
      return 0
    lax.fori_loop(0, NPAD2 // 16, zero_hist, 0)

    def zero_bounce(i, _):
      r = i // 8
      t = i % 8
      bounce[r, pl.ds(t * 16, 16)] = jnp.zeros((16,), jnp.float32)
      return 0
    lax.fori_loop(0, HRT * 8, zero_bounce, 0)

    pltpu.sync_copy(bounce, acc.at[pl.ds(s * HRT, HRT)])

    ones = jnp.ones((16,), jnp.float32)

    def hist_body(j, _):
      idx = dst_v[pl.ds(j * 16, 16)]
      plsc.addupdate_scatter(hist1, [idx], ones)
      return 0
    lax.fori_loop(0, EPT // 16, hist_body, 0)

    def repack(i, _):
      r = i // 8
      t = i % 8
      hist[r, pl.ds(t * 16, 16)] = hist1[pl.ds(r * 128 + t * 16, 16)]
      return 0
    lax.fori_loop(0, HR * 8, repack, 0)

    plsc.subcore_barrier()

    pltpu.sync_copy(hist, acc.at[iota_v.at[0]], add=True)

    plsc.subcore_barrier()

    pltpu.sync_copy(acc.at[pl.ds(s * HRT, HRT)], bounce)
    pltpu.sync_copy(bounce, out_hbm.at[c, pl.ds(s * HRT, HRT)])

  return deg_kernel(dst1, iota2)


def _propagate(g, src2, dst3):
  @functools.partial(
      pl.kernel,
      out_type=(jax.ShapeDtypeStruct((NPAD, D), jnp.float32),
                jax.ShapeDtypeStruct((NPAD, D), jnp.float32)),
      mesh=_sc_mesh(),
      scratch_types=[
          pltpu.VMEM((EPT,), jnp.int32),
          pltpu.VMEM((CH, K), jnp.int32),
          pltpu.VMEM((K, D), jnp.float32),
          pltpu.VMEM((K, D), jnp.float32),
          pltpu.VMEM_SHARED((NPAD, D), jnp.float32),
          pltpu.SemaphoreType.DMA,
          pltpu.SemaphoreType.DMA,
      ],
  )
  def prop_kernel(g_hbm, src_hbm, dst_hbm, out0_hbm, out1_hbm, src_v, dst_v,
                  rows_a, rows_b, acc, sem_a, sem_b):
    c = lax.axis_index("c")
    s = lax.axis_index("s")
    w = s * NC + c
    base = s * RP

    pltpu.async_copy(src_hbm.at[w], src_v, sem_a)
    pltpu.async_copy(dst_hbm.at[w], dst_v, sem_b)
    pltpu.make_async_copy(src_hbm.at[w], src_v, sem_a).wait()
    pltpu.make_async_copy(dst_hbm.at[w], dst_v, sem_b).wait()

    pltpu.async_copy(g_hbm.at[src_v.at[pl.ds(0, K)]], rows_a, sem_a)

    def zero_row(i, _):
      r = i // (D // 16)
      t = i % (D // 16)
      rows_b[r, pl.ds(t * 16, 16)] = jnp.zeros((16,), jnp.float32)
      return 0
    lax.fori_loop(0, CPR * (D // 16), zero_row, 0)
    for i in range(NCP):
      pltpu.async_copy(rows_b, acc.at[pl.ds(base + i * CPR, CPR)], sem_b)
    for i in range(NCP):
      pltpu.make_async_copy(
          rows_b, acc.at[pl.ds(base + i * CPR, CPR)], sem_b).wait()

    plsc.subcore_barrier()

    def chunk2(i, _):
      ja = 2 * i
      jb = 2 * i + 1
      pltpu.async_copy(g_hbm.at[src_v.at[pl.ds(jb * K, K)]], rows_b, sem_b)
      pltpu.make_async_copy(
          g_hbm.at[src_v.at[pl.ds(ja * K, K)]], rows_a, sem_a).wait()
      pltpu.sync_copy(rows_a, acc.at[dst_v.at[ja]], add=True)
      pltpu.async_copy(
          g_hbm.at[src_v.at[pl.ds((ja + 2) * K, K)]], rows_a, sem_a)
      pltpu.make_async_copy(
          g_hbm.at[src_v.at[pl.ds(jb * K, K)]], rows_b, sem_b).wait()
      pltpu.sync_copy(rows_b, acc.at[dst_v.at[jb]], add=True)
      return 0
    lax.fori_loop(0, (CH - 1) // 2, chunk2, 0)

    pltpu.make_async_copy(
        g_hbm.at[src_v.at[pl.ds((CH - 1) * K, K)]], rows_a, sem_a).wait()
    pltpu.sync_copy(rows_a, acc.at[dst_v.at[CH - 1]], add=True)

    plsc.subcore_barrier()

    def copy_out(out_hbm):
      for i in range(NCP):
        buf = rows_a if i % 2 == 0 else rows_b
        sem = sem_a if i % 2 == 0 else sem_b
        sl = pl.ds(base + i * CPR, CPR)
        if i >= 2:
          prev = pl.ds(base + (i - 2) * CPR, CPR)
          pltpu.make_async_copy(buf, out_hbm.at[prev], sem).wait()
        pltpu.sync_copy(acc.at[sl], buf)
        pltpu.async_copy(buf, out_hbm.at[sl], sem)
      for i in range(NCP - 2, NCP):
        buf = rows_a if i % 2 == 0 else rows_b
        sem = sem_a if i % 2 == 0 else sem_b
        sl = pl.ds(base + i * CPR, CPR)
        pltpu.make_async_copy(buf, out_hbm.at[sl], sem).wait()

    @pl.when(c == 0)
    def _():
      copy_out(out0_hbm)

    @pl.when(c != 0)
    def _():
      copy_out(out1_hbm)

  return prop_kernel(g, src2, dst3)


def _row_spec(width):
  return pl.BlockSpec((RB, width), lambda i: (i, 0))


def _full_spec(r, cdim):
  return pl.BlockSpec((r, cdim), lambda i: (0, 0))


def _xw_body(x_ref, w_ref, u_ref):
  u_ref[...] = jnp.dot(x_ref[...], w_ref[...],
                       preferred_element_type=jnp.float32)


def _tc_xw(x, W0):
  return pl.pallas_call(
      _xw_body,
      grid=(NPAD // RB,),
      in_specs=[_row_spec(D), _full_spec(D, D)],
      out_specs=_row_spec(D),
      out_shape=jax.ShapeDtypeStruct((NPAD, D), jnp.float32),
  )(x, W0)


def _scale_body(u_ref, d0_ref, d1_ref, g_ref):
  dinv = lax.rsqrt(d0_ref[...] + d1_ref[...] + 1.0)
  g_ref[...] = dinv * u_ref[...]


def _tc_scale(u, d0, d1):
  return pl.pallas_call(
      _scale_body,
      grid=(NPAD // RB,),
      in_specs=[_row_spec(D), _row_spec(1), _row_spec(1)],
      out_specs=_row_spec(D),
      out_shape=jax.ShapeDtypeStruct((NPAD, D), jnp.float32),
  )(u, d0, d1)


def _g_body(s0_ref, s1_ref, g_in_ref, d0_ref, d1_ref, b_ref, wn_ref,
            g_ref, h_ref):
  dinv = lax.rsqrt(d0_ref[...] + d1_ref[...] + 1.0)
  h = jnp.maximum(
      dinv * (s0_ref[...] + s1_ref[...] + g_in_ref[...]) + b_ref[...], 0.0)
  h_ref[...] = h
  g_ref[...] = dinv * jnp.dot(h, wn_ref[...],
                              preferred_element_type=jnp.float32)


def _tc_g(s0, s1, g_in, d0, d1, b, Wn):
  return pl.pallas_call(
      _g_body,
      grid=(NPAD // RB,),
      in_specs=[_row_spec(D), _row_spec(D), _row_spec(D), _row_spec(1),
                _row_spec(1), _full_spec(1, D), _full_spec(D, D)],
      out_specs=[_row_spec(D), _row_spec(D)],
      out_shape=[jax.ShapeDtypeStruct((NPAD, D), jnp.float32),
                 jax.ShapeDtypeStruct((NPAD, D), jnp.float32)],
  )(s0, s1, g_in, d0, d1, b, Wn)


def _y0_body(h_ref, wh_ref, bl_ref, yo_ref):
  yo_ref[...] = bl_ref[...] + jnp.dot(h_ref[...], wh_ref[...],
                                      preferred_element_type=jnp.float32)


def _tc_y0(h, Wh, bl):
  return pl.pallas_call(
      _y0_body,
      grid=(NPAD // RB,),
      in_specs=[_row_spec(D), _full_spec(D, NCLS), _full_spec(1, NCLS)],
      out_specs=_row_spec(NCLS),
      out_shape=jax.ShapeDtypeStruct((NPAD, NCLS), jnp.float32),
  )(h, Wh, bl)


def _y_body(h_ref, wh_ref, y_ref, yo_ref):
  yo_ref[...] = y_ref[...] + jnp.dot(h_ref[...], wh_ref[...],
                                     preferred_element_type=jnp.float32)


def _tc_y(h, Wh, y):
  return pl.pallas_call(
      _y_body,
      grid=(NPAD // RB,),
      in_specs=[_row_spec(D), _full_spec(D, NCLS), _row_spec(NCLS)],
      out_specs=_row_spec(NCLS),
      out_shape=jax.ShapeDtypeStruct((NPAD, NCLS), jnp.float32),
  )(h, Wh, y)


def _last_body(s0_ref, s1_ref, g_in_ref, d0_ref, d1_ref, b_ref, wh_ref, y_ref,
               yo_ref):
  dinv = lax.rsqrt(d0_ref[...] + d1_ref[...] + 1.0)
  h = jnp.maximum(
      dinv * (s0_ref[...] + s1_ref[...] + g_in_ref[...]) + b_ref[...], 0.0)
  yo_ref[...] = y_ref[...] + jnp.dot(h, wh_ref[...],
                                     preferred_element_type=jnp.float32)


def _lrow_spec(width):
  return pl.BlockSpec((RBL, width), lambda i: (i, 0))


def _tc_last(s0, s1, g_in, d0, d1, b, Wh, y):
  return pl.pallas_call(
      _last_body,
      grid=(N // RBL,),
      in_specs=[_lrow_spec(D), _lrow_spec(D), _lrow_spec(D), _lrow_spec(1),
                _lrow_spec(1), _full_spec(1, D), _full_spec(D, NCLS),
                _lrow_spec(NCLS)],
      out_specs=_lrow_spec(NCLS),
      out_shape=jax.ShapeDtypeStruct((N, NCLS), jnp.float32),
  )(s0, s1, g_in, d0, d1, b, Wh, y)


def kernel(x, edge_index, W0, b0, W1, b1, W2, b2, W3, b3, W_last, b_last):
  src2 = edge_index[0].reshape(NW, EPT)
  dst3 = edge_index[1].reshape(NW, CH, K)
  dst1 = edge_index[1].reshape(NW, EPT)

  iota2 = jnp.arange(HR, dtype=jnp.int32).reshape(1, HR)

  degp = _degree(dst1, iota2)
  degf = degp.reshape(NC, NPAD2)[:, :NPAD]
  d0 = degf[0].reshape(NPAD, 1)
  d1 = degf[1].reshape(NPAD, 1)

  Ws = [W0, W1, W2, W3]
  bs = [b.reshape(1, D) for b in (b0, b1, b2, b3)]
  Whs = [W_last[l * D:(l + 1) * D] for l in range(NL)]

  g = _tc_scale(_tc_xw(x, Ws[0]), d0, d1)
  y = None
  for l in range(NL):
    s0, s1 = _propagate(g, src2, dst3)
    if l < NL - 1:
      g, h = _tc_g(s0, s1, g, d0, d1, bs[l], Ws[l + 1])
      if l == 0:
        y = _tc_y0(h, Whs[l], b_last.reshape(1, NCLS))
      else:
        y = _tc_y(h, Whs[l], y)
    else:
      y = _tc_last(s0, s1, g, d0, d1, bs[l], Whs[l], y)
  return y
